# Initial kernel scaffold; baseline (speedup 1.0000x reference)
import jax
import jax.numpy as jnp
from jax.experimental import pallas as pl


def kernel(x, edge_index, batch, W1, b1, W2, b2, W3, b3, Wl, bl):
    raise NotImplementedError("write your pallas kernel here")


# speedup vs baseline: 6.4618x; 6.4618x over previous
import jax
import jax.numpy as jnp
from jax import lax
from jax.experimental import pallas as pl
from jax.experimental.pallas import tpu as pltpu
from jax.experimental.pallas import tpu_sc as plsc

N, E, DI, DH, DO, G = 10000, 320000, 128, 256, 10, 64
Np = 10240
CH = 128
Ep = 327680
GK = 16
NSUB = 16
NCORE = 2
ROWS_W = Np // NSUB
R = 512
NBLK = Np // R


def _sc_mesh():
    return plsc.VectorSubcoreMesh(core_axis_name="c", subcore_axis_name="s")


def _deg_body(dst3d, zeros, ones, out, acc, didx_v, ones_v):
    c = lax.axis_index("c")
    s = lax.axis_index("s")
    w = c * NSUB + s
    n_groups = dst3d.shape[1] // GK
    r0 = s * ROWS_W
    pltpu.sync_copy(zeros.at[pl.ds(r0, ROWS_W)], acc.at[pl.ds(r0, ROWS_W)])
    pltpu.sync_copy(ones, ones_v)
    plsc.subcore_barrier()

    def group(gk, carry):
        pltpu.sync_copy(dst3d.at[w, pl.ds(gk * GK, GK)], didx_v)

        def chunk(k, c2):
            pltpu.sync_copy(ones_v, acc.at[didx_v.at[k]], add=True)
            return c2

        lax.fori_loop(0, GK, chunk, 0)
        return carry

    lax.fori_loop(0, n_groups, group, 0)
    plsc.subcore_barrier()
    pltpu.sync_copy(acc.at[pl.ds(r0, ROWS_W)], out.at[pl.ds(c * Np + r0, ROWS_W)])


def _deg_call(dst3d, zeros, ones):
    return pl.kernel(
        _deg_body,
        out_type=jax.ShapeDtypeStruct((2 * Np, 128), jnp.float32),
        mesh=_sc_mesh(),
        scratch_types=[
            pltpu.VMEM_SHARED((Np, 128), jnp.float32),
            pltpu.VMEM((GK, CH), jnp.int32),
            pltpu.VMEM((CH, 128), jnp.float32),
        ],
    )(dst3d, zeros, ones)


def _prop_body(gcat, srcb3d, dst3d, out, acc, sidx_v, didx_v, rows_v, gsem):
    c = lax.axis_index("c")
    s = lax.axis_index("s")
    w = c * NSUB + s
    n_groups = dst3d.shape[1] // GK
    r0 = s * ROWS_W
    pltpu.sync_copy(gcat.at[pl.ds(c * Np + r0, ROWS_W)], acc.at[pl.ds(r0, ROWS_W)])
    plsc.subcore_barrier()

    def group(gk, carry):
        pltpu.sync_copy(srcb3d.at[w, pl.ds(gk * GK, GK)], sidx_v)
        pltpu.sync_copy(dst3d.at[s, pl.ds(gk * GK, GK)], didx_v)

        def chunk(k, c2):
            pltpu.async_copy(gcat.at[sidx_v.at[k]], rows_v, gsem).wait()
            pltpu.sync_copy(rows_v, acc.at[didx_v.at[k]], add=True)
            return c2

        lax.fori_loop(0, GK, chunk, 0)
        return carry

    lax.fori_loop(0, n_groups, group, 0)
    plsc.subcore_barrier()
    pltpu.sync_copy(acc.at[pl.ds(r0, ROWS_W)], out.at[pl.ds(c * Np + r0, ROWS_W)])


def _prop_call(gcat, srcb3d, dst3d):
    return pl.kernel(
        _prop_body,
        out_type=jax.ShapeDtypeStruct((2 * Np, DH // 2), jnp.float32),
        mesh=_sc_mesh(),
        scratch_types=[
            pltpu.VMEM_SHARED((Np, DH // 2), jnp.float32),
            pltpu.VMEM((GK, CH), jnp.int32),
            pltpu.VMEM((GK, CH), jnp.int32),
            pltpu.VMEM((CH, DH // 2), jnp.float32),
            pltpu.SemaphoreType.DMA,
        ],
    )(gcat, srcb3d, dst3d)


def _first_mm_body(x_ref, w_ref, dinv_ref, out_ref):
    j = pl.program_id(1)
    g = lax.dot_general(x_ref[...], w_ref[...], (((1,), (0,)), ((), ())),
                        preferred_element_type=jnp.float32)
    g = g * dinv_ref[...]
    half = jnp.where(j == 0, g[:, : DH // 2], g[:, DH // 2 :])
    out_ref[...] = half


def _first_mm(x, W1, dinv):
    return pl.pallas_call(
        _first_mm_body,
        grid=(NBLK, 2),
        in_specs=[
            pl.BlockSpec((R, DI), lambda i, j: (i, 0)),
            pl.BlockSpec((DI, DH), lambda i, j: (0, 0)),
            pl.BlockSpec((R, 1), lambda i, j: (i, 0)),
        ],
        out_specs=pl.BlockSpec((R, DH // 2), lambda i, j: (j * NBLK + i, 0)),
        out_shape=jax.ShapeDtypeStruct((2 * Np, DH // 2), jnp.float32),
    )(x, W1, dinv)


def _mid_mm_body(sa_ref, sb_ref, dinv_ref, b_ref, w_ref, out_ref):
    j = pl.program_id(1)
    dinv = dinv_ref[...]
    sfull = jnp.concatenate([sa_ref[...], sb_ref[...]], axis=1)
    h = jnp.maximum(sfull * dinv + b_ref[...], 0.0)
    g = lax.dot_general(h, w_ref[...], (((1,), (0,)), ((), ())),
                        preferred_element_type=jnp.float32)
    g = g * dinv
    half = jnp.where(j == 0, g[:, : DH // 2], g[:, DH // 2 :])
    out_ref[...] = half


def _mid_mm(s, dinv, b, W):
    return pl.pallas_call(
        _mid_mm_body,
        grid=(NBLK, 2),
        in_specs=[
            pl.BlockSpec((R, DH // 2), lambda i, j: (i, 0)),
            pl.BlockSpec((R, DH // 2), lambda i, j: (i + NBLK, 0)),
            pl.BlockSpec((R, 1), lambda i, j: (i, 0)),
            pl.BlockSpec((1, DH), lambda i, j: (0, 0)),
            pl.BlockSpec((DH, DH), lambda i, j: (0, 0)),
        ],
        out_specs=pl.BlockSpec((R, DH // 2), lambda i, j: (j * NBLK + i, 0)),
        out_shape=jax.ShapeDtypeStruct((2 * Np, DH // 2), jnp.float32),
    )(s, s, dinv, b, W)


def _final_body(sa_ref, sb_ref, dinv_ref, b_ref, batch_ref, wl_ref, bl_ref,
                out_ref, hacc, cacc):
    i = pl.program_id(0)

    @pl.when(i == 0)
    def _init():
        hacc[...] = jnp.zeros_like(hacc)
        cacc[...] = jnp.zeros_like(cacc)

    sfull = jnp.concatenate([sa_ref[...], sb_ref[...]], axis=1)
    h3 = sfull * dinv_ref[...] + b_ref[...]
    lab = batch_ref[...]
    oh = (lab == lax.broadcasted_iota(jnp.int32, (R, G), 1)).astype(jnp.float32)
    hacc[...] += lax.dot_general(oh, h3, (((0,), (0,)), ((), ())),
                                 preferred_element_type=jnp.float32)
    cacc[...] += lax.dot_general(oh, jnp.ones((R, DH), jnp.float32),
                                 (((0,), (0,)), ((), ())),
                                 preferred_element_type=jnp.float32)

    @pl.when(i == pl.num_programs(0) - 1)
    def _fin():
        hg = hacc[...] / jnp.maximum(cacc[...], 1.0)
        o = lax.dot_general(hg, wl_ref[...], (((1,), (0,)), ((), ())),
                            preferred_element_type=jnp.float32) + bl_ref[...]
        m = jnp.max(o, axis=1, keepdims=True)
        e = jnp.exp(o - m)
        out_ref[...] = e / jnp.sum(e, axis=1, keepdims=True)


def _final(s, dinv, b3, batch2d, Wl, bl):
    return pl.pallas_call(
        _final_body,
        grid=(NBLK,),
        in_specs=[
            pl.BlockSpec((R, DH // 2), lambda i: (i, 0)),
            pl.BlockSpec((R, DH // 2), lambda i: (i + NBLK, 0)),
            pl.BlockSpec((R, 1), lambda i: (i, 0)),
            pl.BlockSpec((1, DH), lambda i: (0, 0)),
            pl.BlockSpec((R, 1), lambda i: (i, 0)),
            pl.BlockSpec((DH, DO), lambda i: (0, 0)),
            pl.BlockSpec((1, DO), lambda i: (0, 0)),
        ],
        out_specs=pl.BlockSpec((G, DO), lambda i: (0, 0)),
        out_shape=jax.ShapeDtypeStruct((G, DO), jnp.float32),
        scratch_shapes=[
            pltpu.VMEM((G, DH), jnp.float32),
            pltpu.VMEM((G, DH), jnp.float32),
        ],
    )(s, s, dinv, b3, batch2d, Wl, bl)


def kernel(x, edge_index, batch, W1, b1, W2, b2, W3, b3, Wl, bl):
    src = jnp.pad(edge_index[0], (0, Ep - E))
    dst = jnp.pad(edge_index[1], (0, Ep - E), constant_values=N)
    srcb3d = jnp.concatenate([src, src + Np]).reshape(NCORE * NSUB, -1, CH)
    dst3d_p = dst.reshape(NSUB, -1, CH)
    dst3d_d = dst.reshape(NCORE * NSUB, -1, CH)
    zeros8 = jnp.zeros((Np, 128), jnp.float32)
    ones8 = jnp.ones((CH, 128), jnp.float32)

    dpart = _deg_call(dst3d_d, zeros8, ones8)
    deg = 1.0 + dpart[:N, 0] + dpart[Np : Np + N, 0]
    dinv = jnp.pad(lax.rsqrt(deg), (0, Np - N), constant_values=1.0)[:, None]

    xp = jnp.pad(x, ((0, Np - N), (0, 0)))
    batch2d = jnp.pad(batch, (0, Np - N), constant_values=G)[:, None]

    b1r = b1[None, :]
    b2r = b2[None, :]
    b3r = b3[None, :]
    blr = bl[None, :]

    g = _first_mm(xp, W1, dinv)
    s1 = _prop_call(g, srcb3d, dst3d_p)
    g = _mid_mm(s1, dinv, b1r, W2)
    s2 = _prop_call(g, srcb3d, dst3d_p)
    g = _mid_mm(s2, dinv, b2r, W3)
    s3 = _prop_call(g, srcb3d, dst3d_p)
    return _final(s3, dinv, b3r, batch2d, Wl, blr)


# speedup vs baseline: 7.3310x; 1.1345x over previous
import jax
import jax.numpy as jnp
from jax import lax
from jax.experimental import pallas as pl
from jax.experimental.pallas import tpu as pltpu
from jax.experimental.pallas import tpu_sc as plsc

N, E, DI, DH, DO, G = 10000, 320000, 128, 256, 10, 64
Np = 10240
CH = 128
Ep = 327680
GK = 16
NBUF = 2
NSUB = 16
NCORE = 2
ROWS_W = Np // NSUB
R = 512
NBLK = Np // R


def _sc_mesh():
    return plsc.VectorSubcoreMesh(core_axis_name="c", subcore_axis_name="s")


def _deg_body(dst3d, zeros, ones, out, acc, didx_v, ones_v):
    c = lax.axis_index("c")
    s = lax.axis_index("s")
    w = c * NSUB + s
    n_groups = dst3d.shape[1] // GK
    r0 = s * ROWS_W
    pltpu.sync_copy(zeros.at[pl.ds(r0, ROWS_W)], acc.at[pl.ds(r0, ROWS_W)])
    pltpu.sync_copy(ones, ones_v)
    plsc.subcore_barrier()

    def group(gk, carry):
        pltpu.sync_copy(dst3d.at[w, pl.ds(gk * GK, GK)], didx_v)

        def chunk(k, c2):
            pltpu.sync_copy(ones_v, acc.at[didx_v.at[k]], add=True)
            return c2

        lax.fori_loop(0, GK, chunk, 0)
        return carry

    lax.fori_loop(0, n_groups, group, 0)
    plsc.subcore_barrier()
    pltpu.sync_copy(acc.at[pl.ds(r0, ROWS_W)], out.at[pl.ds(c * Np + r0, ROWS_W)])


def _deg_call(dst3d, zeros, ones):
    return pl.kernel(
        _deg_body,
        out_type=jax.ShapeDtypeStruct((2 * Np, 128), jnp.float32),
        mesh=_sc_mesh(),
        scratch_types=[
            pltpu.VMEM_SHARED((Np, 128), jnp.float32),
            pltpu.VMEM((GK, CH), jnp.int32),
            pltpu.VMEM((CH, 128), jnp.float32),
        ],
    )(dst3d, zeros, ones)


def _prop_body(gcat, gfull, srcb3d, dst3d, out, acc, gs, sidx_v, didx_v, rows,
               gsems, ssem):
    c = lax.axis_index("c")
    s = lax.axis_index("s")
    w = c * NSUB + s
    n_groups = dst3d.shape[1] // GK
    r0 = s * ROWS_W
    pltpu.sync_copy(gcat.at[pl.ds(c * Np + r0, ROWS_W)], acc.at[pl.ds(r0, ROWS_W)])
    plsc.subcore_barrier()

    def group(gk, carry):
        pltpu.sync_copy(srcb3d.at[w, pl.ds(gk * GK, GK)], sidx_v)
        pltpu.sync_copy(dst3d.at[s, pl.ds(gk * GK, GK)], didx_v)
        g_descs = [None] * NBUF
        for k in range(NBUF - 1):
            g_descs[k] = pltpu.async_copy(
                gcat.at[sidx_v.at[k]], rows[k], gsems[k])
        for k in range(GK):
            b = k % NBUF
            if k + NBUF - 1 < GK:
                nb = (k + NBUF - 1) % NBUF
                g_descs[nb] = pltpu.async_copy(
                    gcat.at[sidx_v.at[k + NBUF - 1]], rows[nb], gsems[nb])
            g_descs[b].wait()
            pltpu.sync_copy(rows[b], acc.at[didx_v.at[k]], add=True)
        return carry

    lax.fori_loop(0, n_groups, group, 0)
    plsc.subcore_barrier()
    pltpu.sync_copy(acc.at[pl.ds(r0, ROWS_W)], out.at[pl.ds(c * Np + r0, ROWS_W)])


def _prop_call(gcat, srcb3d, dst3d):
    def body(gcat_r, gfull_r, srcb_r, dst_r, out_r, sidx_v, didx_v, acc, gs,
             *bufs):
        rows = bufs[:NBUF]
        gsems = bufs[NBUF:2 * NBUF]
        ssem = bufs[2 * NBUF]
        _prop_body(gcat_r, gfull_r, srcb_r, dst_r, out_r, acc, gs, sidx_v,
                   didx_v, rows, gsems, ssem)

    gfull = gcat.reshape(Np, 2 * (DH // 2))
    return pl.kernel(
        body,
        out_type=jax.ShapeDtypeStruct((2 * Np, DH // 2), jnp.float32),
        mesh=_sc_mesh(),
        scratch_types=[
            pltpu.VMEM((GK, CH), jnp.int32),
            pltpu.VMEM((GK, CH), jnp.int32),
            pltpu.VMEM_SHARED((Np, DH // 2), jnp.float32),
            pltpu.VMEM_SHARED((8, DH // 2), jnp.float32),
        ] + [pltpu.VMEM((CH, DH // 2), jnp.float32) for _ in range(NBUF)]
          + [pltpu.SemaphoreType.DMA for _ in range(NBUF + 1)],
    )(gcat, gfull, srcb3d, dst3d)


def _first_mm_body(x_ref, w_ref, dinv_ref, out_ref):
    j = pl.program_id(1)
    g = lax.dot_general(x_ref[...], w_ref[...], (((1,), (0,)), ((), ())),
                        preferred_element_type=jnp.float32)
    g = g * dinv_ref[...]
    half = jnp.where(j == 0, g[:, : DH // 2], g[:, DH // 2 :])
    out_ref[...] = half


def _first_mm(x, W1, dinv):
    return pl.pallas_call(
        _first_mm_body,
        grid=(NBLK, 2),
        in_specs=[
            pl.BlockSpec((R, DI), lambda i, j: (i, 0)),
            pl.BlockSpec((DI, DH), lambda i, j: (0, 0)),
            pl.BlockSpec((R, 1), lambda i, j: (i, 0)),
        ],
        out_specs=pl.BlockSpec((R, DH // 2), lambda i, j: (j * NBLK + i, 0)),
        out_shape=jax.ShapeDtypeStruct((2 * Np, DH // 2), jnp.float32),
    )(x, W1, dinv)


def _mid_mm_body(sa_ref, sb_ref, dinv_ref, b_ref, w_ref, out_ref):
    j = pl.program_id(1)
    dinv = dinv_ref[...]
    sfull = jnp.concatenate([sa_ref[...], sb_ref[...]], axis=1)
    h = jnp.maximum(sfull * dinv + b_ref[...], 0.0)
    g = lax.dot_general(h, w_ref[...], (((1,), (0,)), ((), ())),
                        preferred_element_type=jnp.float32)
    g = g * dinv
    half = jnp.where(j == 0, g[:, : DH // 2], g[:, DH // 2 :])
    out_ref[...] = half


def _mid_mm(s, dinv, b, W):
    return pl.pallas_call(
        _mid_mm_body,
        grid=(NBLK, 2),
        in_specs=[
            pl.BlockSpec((R, DH // 2), lambda i, j: (i, 0)),
            pl.BlockSpec((R, DH // 2), lambda i, j: (i + NBLK, 0)),
            pl.BlockSpec((R, 1), lambda i, j: (i, 0)),
            pl.BlockSpec((1, DH), lambda i, j: (0, 0)),
            pl.BlockSpec((DH, DH), lambda i, j: (0, 0)),
        ],
        out_specs=pl.BlockSpec((R, DH // 2), lambda i, j: (j * NBLK + i, 0)),
        out_shape=jax.ShapeDtypeStruct((2 * Np, DH // 2), jnp.float32),
    )(s, s, dinv, b, W)


def _final_body(sa_ref, sb_ref, dinv_ref, b_ref, batch_ref, wl_ref, bl_ref,
                out_ref, hacc, cacc):
    i = pl.program_id(0)

    @pl.when(i == 0)
    def _init():
        hacc[...] = jnp.zeros_like(hacc)
        cacc[...] = jnp.zeros_like(cacc)

    sfull = jnp.concatenate([sa_ref[...], sb_ref[...]], axis=1)
    h3 = sfull * dinv_ref[...] + b_ref[...]
    lab = batch_ref[...]
    oh = (lab == lax.broadcasted_iota(jnp.int32, (R, G), 1)).astype(jnp.float32)
    hacc[...] += lax.dot_general(oh, h3, (((0,), (0,)), ((), ())),
                                 preferred_element_type=jnp.float32)
    cacc[...] += lax.dot_general(oh, jnp.ones((R, DH), jnp.float32),
                                 (((0,), (0,)), ((), ())),
                                 preferred_element_type=jnp.float32)

    @pl.when(i == pl.num_programs(0) - 1)
    def _fin():
        hg = hacc[...] / jnp.maximum(cacc[...], 1.0)
        o = lax.dot_general(hg, wl_ref[...], (((1,), (0,)), ((), ())),
                            preferred_element_type=jnp.float32) + bl_ref[...]
        m = jnp.max(o, axis=1, keepdims=True)
        e = jnp.exp(o - m)
        out_ref[...] = e / jnp.sum(e, axis=1, keepdims=True)


def _final(s, dinv, b3, batch2d, Wl, bl):
    return pl.pallas_call(
        _final_body,
        grid=(NBLK,),
        in_specs=[
            pl.BlockSpec((R, DH // 2), lambda i: (i, 0)),
            pl.BlockSpec((R, DH // 2), lambda i: (i + NBLK, 0)),
            pl.BlockSpec((R, 1), lambda i: (i, 0)),
            pl.BlockSpec((1, DH), lambda i: (0, 0)),
            pl.BlockSpec((R, 1), lambda i: (i, 0)),
            pl.BlockSpec((DH, DO), lambda i: (0, 0)),
            pl.BlockSpec((1, DO), lambda i: (0, 0)),
        ],
        out_specs=pl.BlockSpec((G, DO), lambda i: (0, 0)),
        out_shape=jax.ShapeDtypeStruct((G, DO), jnp.float32),
        scratch_shapes=[
            pltpu.VMEM((G, DH), jnp.float32),
            pltpu.VMEM((G, DH), jnp.float32),
        ],
    )(s, s, dinv, b3, batch2d, Wl, bl)


def kernel(x, edge_index, batch, W1, b1, W2, b2, W3, b3, Wl, bl):
    src = jnp.pad(edge_index[0], (0, Ep - E))
    dst = jnp.pad(edge_index[1], (0, Ep - E), constant_values=N)
    srcb3d = jnp.concatenate([src, src + Np]).reshape(NCORE * NSUB, -1, CH)
    dst3d_p = dst.reshape(NSUB, -1, CH)
    dst3d_d = dst.reshape(NCORE * NSUB, -1, CH)
    zeros8 = jnp.zeros((Np, 128), jnp.float32)
    ones8 = jnp.ones((CH, 128), jnp.float32)

    dpart = _deg_call(dst3d_d, zeros8, ones8)
    deg = 1.0 + dpart[:N, 0] + dpart[Np : Np + N, 0]
    dinv = jnp.pad(lax.rsqrt(deg), (0, Np - N), constant_values=1.0)[:, None]

    xp = jnp.pad(x, ((0, Np - N), (0, 0)))
    batch2d = jnp.pad(batch, (0, Np - N), constant_values=G)[:, None]

    b1r = b1[None, :]
    b2r = b2[None, :]
    b3r = b3[None, :]
    blr = bl[None, :]

    g = _first_mm(xp, W1, dinv)
    s1 = _prop_call(g, srcb3d, dst3d_p)
    g = _mid_mm(s1, dinv, b1r, W2)
    s2 = _prop_call(g, srcb3d, dst3d_p)
    g = _mid_mm(s2, dinv, b2r, W3)
    s3 = _prop_call(g, srcb3d, dst3d_p)
    return _final(s3, dinv, b3r, batch2d, Wl, blr)


# speedup vs baseline: 7.6015x; 1.0369x over previous
import jax
import jax.numpy as jnp
from jax import lax
from jax.experimental import pallas as pl
from jax.experimental.pallas import tpu as pltpu
from jax.experimental.pallas import tpu_sc as plsc

N, E, DI, DH, DO, G = 10000, 320000, 128, 256, 10, 64
Np = 10240
CH = 128
Ep = 327680
GK = 16
NBUF = 2
NSUB = 16
NCORE = 2
ROWS_W = Np // NSUB
R = 512
NBLK = Np // R


def _sc_mesh():
    return plsc.VectorSubcoreMesh(core_axis_name="c", subcore_axis_name="s")


def _deg_body(dst3d, zeros, ones, out, acc, didx_v, ones_v):
    c = lax.axis_index("c")
    s = lax.axis_index("s")
    w = c * NSUB + s
    n_groups = dst3d.shape[1] // GK
    r0 = s * ROWS_W
    pltpu.sync_copy(zeros.at[pl.ds(r0, ROWS_W)], acc.at[pl.ds(r0, ROWS_W)])
    pltpu.sync_copy(ones, ones_v)
    plsc.subcore_barrier()

    def group(gk, carry):
        pltpu.sync_copy(dst3d.at[w, pl.ds(gk * GK, GK)], didx_v)

        def chunk(k, c2):
            pltpu.sync_copy(ones_v, acc.at[didx_v.at[k]], add=True)
            return c2

        lax.fori_loop(0, GK, chunk, 0)
        return carry

    lax.fori_loop(0, n_groups, group, 0)
    plsc.subcore_barrier()
    pltpu.sync_copy(acc.at[pl.ds(r0, ROWS_W)], out.at[pl.ds(c * Np + r0, ROWS_W)])


def _deg_call(dst3d, zeros, ones):
    return pl.kernel(
        _deg_body,
        out_type=jax.ShapeDtypeStruct((2 * Np, 128), jnp.float32),
        mesh=_sc_mesh(),
        scratch_types=[
            pltpu.VMEM_SHARED((Np, 128), jnp.float32),
            pltpu.VMEM((GK, CH), jnp.int32),
            pltpu.VMEM((CH, 128), jnp.float32),
        ],
    )(dst3d, zeros, ones)


def _prop_body(gcat, srcb3d, dst3d, out, acc, sidx_v, didx_v, rows,
               gsems, ssem):
    c = lax.axis_index("c")
    s = lax.axis_index("s")
    w = c * NSUB + s
    n_groups = dst3d.shape[1] // GK
    r0 = s * ROWS_W
    pltpu.sync_copy(gcat.at[pl.ds(c * Np + r0, ROWS_W)], acc.at[pl.ds(r0, ROWS_W)])
    plsc.subcore_barrier()

    def group(gk, carry):
        pltpu.sync_copy(srcb3d.at[w, pl.ds(gk * GK, GK)], sidx_v)
        pltpu.sync_copy(dst3d.at[s, pl.ds(gk * GK, GK)], didx_v)
        g_descs = [None] * NBUF
        for k in range(NBUF - 1):
            g_descs[k] = pltpu.async_copy(
                gcat.at[sidx_v.at[k]], rows[k], gsems[k])
        for k in range(GK):
            b = k % NBUF
            if k + NBUF - 1 < GK:
                nb = (k + NBUF - 1) % NBUF
                g_descs[nb] = pltpu.async_copy(
                    gcat.at[sidx_v.at[k + NBUF - 1]], rows[nb], gsems[nb])
            g_descs[b].wait()
            pltpu.sync_copy(rows[b], acc.at[didx_v.at[k]], add=True)
        return carry

    lax.fori_loop(0, n_groups, group, 0)
    plsc.subcore_barrier()
    pltpu.sync_copy(acc.at[pl.ds(r0, ROWS_W)], out.at[pl.ds(c * Np + r0, ROWS_W)])


def _prop_call(gcat, srcb3d, dst3d):
    def body(gcat_r, srcb_r, dst_r, out_r, sidx_v, didx_v, acc, *bufs):
        rows = bufs[:NBUF]
        gsems = bufs[NBUF:2 * NBUF]
        ssem = bufs[2 * NBUF]
        _prop_body(gcat_r, srcb_r, dst_r, out_r, acc, sidx_v,
                   didx_v, rows, gsems, ssem)

    return pl.kernel(
        body,
        out_type=jax.ShapeDtypeStruct((2 * Np, DH // 2), jnp.float32),
        mesh=_sc_mesh(),
        scratch_types=[
            pltpu.VMEM((GK, CH), jnp.int32),
            pltpu.VMEM((GK, CH), jnp.int32),
            pltpu.VMEM_SHARED((Np, DH // 2), jnp.float32),
        ] + [pltpu.VMEM((CH, DH // 2), jnp.float32) for _ in range(NBUF)]
          + [pltpu.SemaphoreType.DMA for _ in range(NBUF + 1)],
    )(gcat, srcb3d, dst3d)


def _first_mm_body(x_ref, w_ref, dinv_ref, out_ref):
    j = pl.program_id(1)
    g = lax.dot_general(x_ref[...], w_ref[...], (((1,), (0,)), ((), ())),
                        preferred_element_type=jnp.float32)
    g = g * dinv_ref[...]
    half = jnp.where(j == 0, g[:, : DH // 2], g[:, DH // 2 :])
    out_ref[...] = half


def _first_mm(x, W1, dinv):
    return pl.pallas_call(
        _first_mm_body,
        grid=(NBLK, 2),
        in_specs=[
            pl.BlockSpec((R, DI), lambda i, j: (i, 0)),
            pl.BlockSpec((DI, DH), lambda i, j: (0, 0)),
            pl.BlockSpec((R, 1), lambda i, j: (i, 0)),
        ],
        out_specs=pl.BlockSpec((R, DH // 2), lambda i, j: (j * NBLK + i, 0)),
        out_shape=jax.ShapeDtypeStruct((2 * Np, DH // 2), jnp.float32),
    )(x, W1, dinv)


def _mid_mm_body(sa_ref, sb_ref, dinv_ref, b_ref, w_ref, out_ref):
    j = pl.program_id(1)
    dinv = dinv_ref[...]
    sfull = jnp.concatenate([sa_ref[...], sb_ref[...]], axis=1)
    h = jnp.maximum(sfull * dinv + b_ref[...], 0.0)
    g = lax.dot_general(h, w_ref[...], (((1,), (0,)), ((), ())),
                        preferred_element_type=jnp.float32)
    g = g * dinv
    half = jnp.where(j == 0, g[:, : DH // 2], g[:, DH // 2 :])
    out_ref[...] = half


def _mid_mm(s, dinv, b, W):
    return pl.pallas_call(
        _mid_mm_body,
        grid=(NBLK, 2),
        in_specs=[
            pl.BlockSpec((R, DH // 2), lambda i, j: (i, 0)),
            pl.BlockSpec((R, DH // 2), lambda i, j: (i + NBLK, 0)),
            pl.BlockSpec((R, 1), lambda i, j: (i, 0)),
            pl.BlockSpec((1, DH), lambda i, j: (0, 0)),
            pl.BlockSpec((DH, DH), lambda i, j: (0, 0)),
        ],
        out_specs=pl.BlockSpec((R, DH // 2), lambda i, j: (j * NBLK + i, 0)),
        out_shape=jax.ShapeDtypeStruct((2 * Np, DH // 2), jnp.float32),
    )(s, s, dinv, b, W)


def _final_body(sa_ref, sb_ref, dinv_ref, b_ref, batch_ref, wl_ref, bl_ref,
                out_ref, hacc, cacc):
    i = pl.program_id(0)

    @pl.when(i == 0)
    def _init():
        hacc[...] = jnp.zeros_like(hacc)
        cacc[...] = jnp.zeros_like(cacc)

    sfull = jnp.concatenate([sa_ref[...], sb_ref[...]], axis=1)
    h3 = sfull * dinv_ref[...] + b_ref[...]
    lab = batch_ref[...]
    oh = (lab == lax.broadcasted_iota(jnp.int32, (R, G), 1)).astype(jnp.float32)
    hacc[...] += lax.dot_general(oh, h3, (((0,), (0,)), ((), ())),
                                 preferred_element_type=jnp.float32)
    cacc[...] += lax.dot_general(oh, jnp.ones((R, DH), jnp.float32),
                                 (((0,), (0,)), ((), ())),
                                 preferred_element_type=jnp.float32)

    @pl.when(i == pl.num_programs(0) - 1)
    def _fin():
        hg = hacc[...] / jnp.maximum(cacc[...], 1.0)
        o = lax.dot_general(hg, wl_ref[...], (((1,), (0,)), ((), ())),
                            preferred_element_type=jnp.float32) + bl_ref[...]
        m = jnp.max(o, axis=1, keepdims=True)
        e = jnp.exp(o - m)
        out_ref[...] = e / jnp.sum(e, axis=1, keepdims=True)


def _final(s, dinv, b3, batch2d, Wl, bl):
    return pl.pallas_call(
        _final_body,
        grid=(NBLK,),
        in_specs=[
            pl.BlockSpec((R, DH // 2), lambda i: (i, 0)),
            pl.BlockSpec((R, DH // 2), lambda i: (i + NBLK, 0)),
            pl.BlockSpec((R, 1), lambda i: (i, 0)),
            pl.BlockSpec((1, DH), lambda i: (0, 0)),
            pl.BlockSpec((R, 1), lambda i: (i, 0)),
            pl.BlockSpec((DH, DO), lambda i: (0, 0)),
            pl.BlockSpec((1, DO), lambda i: (0, 0)),
        ],
        out_specs=pl.BlockSpec((G, DO), lambda i: (0, 0)),
        out_shape=jax.ShapeDtypeStruct((G, DO), jnp.float32),
        scratch_shapes=[
            pltpu.VMEM((G, DH), jnp.float32),
            pltpu.VMEM((G, DH), jnp.float32),
        ],
    )(s, s, dinv, b3, batch2d, Wl, bl)


def kernel(x, edge_index, batch, W1, b1, W2, b2, W3, b3, Wl, bl):
    src = jnp.pad(edge_index[0], (0, Ep - E))
    dst = jnp.pad(edge_index[1], (0, Ep - E), constant_values=N)
    srcb3d = jnp.concatenate([src, src + Np]).reshape(NCORE * NSUB, -1, CH)
    dst3d_p = dst.reshape(NSUB, -1, CH)
    dst3d_d = dst.reshape(NCORE * NSUB, -1, CH)
    zeros8 = jnp.zeros((Np, 128), jnp.float32)
    ones8 = jnp.ones((CH, 128), jnp.float32)

    dpart = _deg_call(dst3d_d, zeros8, ones8)
    deg = 1.0 + dpart[:N, 0] + dpart[Np : Np + N, 0]
    dinv = jnp.pad(lax.rsqrt(deg), (0, Np - N), constant_values=1.0)[:, None]

    xp = jnp.pad(x, ((0, Np - N), (0, 0)))
    batch2d = jnp.pad(batch, (0, Np - N), constant_values=G)[:, None]

    b1r = b1[None, :]
    b2r = b2[None, :]
    b3r = b3[None, :]
    blr = bl[None, :]

    g = _first_mm(xp, W1, dinv)
    s1 = _prop_call(g, srcb3d, dst3d_p)
    g = _mid_mm(s1, dinv, b1r, W2)
    s2 = _prop_call(g, srcb3d, dst3d_p)
    g = _mid_mm(s2, dinv, b2r, W3)
    s3 = _prop_call(g, srcb3d, dst3d_p)
    return _final(s3, dinv, b3r, batch2d, Wl, blr)
